# baseline (speedup 1.0000x reference)
import functools

import jax
import jax.numpy as jnp
from jax import lax
from jax.experimental import pallas as pl
from jax.experimental.pallas import tpu as pltpu
from jax.experimental.pallas import tpu_sc as plsc

IN_DIM = 128
OUT_DIM = 32
PAD_DIM = 128
N_NODES = 10000
N_INST = 320000
ALPHA = 0.2

NUM_CORES = 2
NUM_SUBCORES = 16
NW = NUM_CORES * NUM_SUBCORES
CHUNK = 128
NCHUNK_TOTAL = N_INST // CHUNK


def _tables_body(dis_ref, chem_ref, pd_ref, pc_ref, w_ref, g0_ref, g1_ref, g2_ref):
    q0 = jnp.dot(pd_ref[...], w_ref[0:32, :], preferred_element_type=jnp.float32)
    q1 = jnp.dot(pd_ref[...], w_ref[32:64, :], preferred_element_type=jnp.float32)
    q2 = jnp.dot(pc_ref[...], w_ref[64:96, :], preferred_element_type=jnp.float32)
    d = dis_ref[...]
    c = chem_ref[...]
    z = jnp.zeros((d.shape[0], PAD_DIM - OUT_DIM), jnp.float32)
    g0_ref[...] = jnp.concatenate(
        [jnp.dot(d, q0, preferred_element_type=jnp.float32), z], axis=1)
    g1_ref[...] = jnp.concatenate(
        [jnp.dot(d, q1, preferred_element_type=jnp.float32), z], axis=1)
    g2_ref[...] = jnp.concatenate(
        [jnp.dot(c, q2, preferred_element_type=jnp.float32), z], axis=1)


def _make_tables(disease_feats, chemical_feats, P_disease, P_chemical, W_CDD):
    blk = 2000
    grid = N_NODES // blk
    return pl.pallas_call(
        _tables_body,
        grid=(grid,),
        in_specs=[
            pl.BlockSpec((blk, IN_DIM), lambda i: (i, 0)),
            pl.BlockSpec((blk, IN_DIM), lambda i: (i, 0)),
            pl.BlockSpec((IN_DIM, OUT_DIM), lambda i: (0, 0)),
            pl.BlockSpec((IN_DIM, OUT_DIM), lambda i: (0, 0)),
            pl.BlockSpec((3 * OUT_DIM, OUT_DIM), lambda i: (0, 0)),
        ],
        out_specs=[
            pl.BlockSpec((blk, PAD_DIM), lambda i: (i, 0)),
            pl.BlockSpec((blk, PAD_DIM), lambda i: (i, 0)),
            pl.BlockSpec((blk, PAD_DIM), lambda i: (i, 0)),
        ],
        out_shape=[jax.ShapeDtypeStruct((N_NODES, PAD_DIM), jnp.float32)] * 3,
    )(disease_feats, chemical_feats, P_disease, P_chemical, W_CDD)


def _sc_body(g0_hbm, g1_hbm, g2_hbm, i0_hbm, i1_hbm, i2_hbm, out_hbm,
             i_v, r_v, o_v, isem, gsem, osem):
    wid = lax.axis_index("s") * NUM_CORES + lax.axis_index("c")
    lo = wid * NCHUNK_TOTAL // NW
    hi = (wid + 1) * NCHUNK_TOTAL // NW
    n = hi - lo
    tables = (g0_hbm, g1_hbm, g2_hbm)
    idxs = (i0_hbm, i1_hbm, i2_hbm)

    def load_idx(chunk, slot):
        for k in range(3):
            pltpu.async_copy(
                idxs[k].at[pl.ds(chunk * CHUNK, CHUNK)], i_v.at[slot, k], isem)

    def wait_idx(slot):
        for k in range(3):
            pltpu.make_async_copy(
                idxs[k].at[pl.ds(0, CHUNK)], i_v.at[slot, k], isem).wait()

    def issue_gathers(slot):
        for k in range(3):
            pltpu.async_copy(tables[k].at[i_v.at[slot, k]], r_v.at[slot, k], gsem)

    def wait_gathers(slot):
        for k in range(3):
            pltpu.make_async_copy(
                tables[k].at[i_v.at[slot, k]], r_v.at[slot, k], gsem).wait()

    def compute(slot):
        @plsc.parallel_loop(0, CHUNK, 1, unroll=4)
        def _(r):
            for h in (0, 1):
                s = pl.ds(h * 16, 16)
                x = r_v[slot, 0, r, s] + r_v[slot, 1, r, s] + r_v[slot, 2, r, s]
                o_v[r, s] = jnp.maximum(x, ALPHA * x)

    def issue_store(chunk):
        pltpu.async_copy(o_v, out_hbm.at[pl.ds(chunk * CHUNK, CHUNK)], osem)

    def wait_store():
        pltpu.make_async_copy(o_v, out_hbm.at[pl.ds(0, CHUNK)], osem).wait()

    load_idx(lo, 0)
    wait_idx(0)
    issue_gathers(0)
    load_idx(lo + 1, 1)

    def step(j, carry):
        slot = lax.rem(j, 2)
        nslot = 1 - slot
        wait_idx(nslot)
        issue_gathers(nslot)
        wait_gathers(slot)

        @pl.when(j < n - 2)
        def _():
            load_idx(lo + j + 2, slot)

        @pl.when(j > 0)
        def _():
            wait_store()
        compute(slot)
        issue_store(lo + j)
        return carry

    lax.fori_loop(0, n - 1, step, 0)

    lslot = lax.rem(n - 1, 2)
    wait_gathers(lslot)
    wait_store()
    compute(lslot)
    issue_store(hi - 1)
    wait_store()


@functools.cache
def _sc_gather():
    return functools.partial(
        pl.kernel,
        out_type=jax.ShapeDtypeStruct((N_INST, OUT_DIM), jnp.float32),
        mesh=plsc.VectorSubcoreMesh(
            core_axis_name="c", subcore_axis_name="s",
            num_cores=NUM_CORES, num_subcores=NUM_SUBCORES),
        scratch_types=[
            pltpu.VMEM((2, 3, CHUNK), jnp.int32),
            pltpu.VMEM((2, 3, CHUNK, PAD_DIM), jnp.float32),
            pltpu.VMEM((CHUNK, OUT_DIM), jnp.float32),
            pltpu.SemaphoreType.DMA,
            pltpu.SemaphoreType.DMA,
            pltpu.SemaphoreType.DMA,
        ],
        compiler_params=pltpu.CompilerParams(use_tc_tiling_on_sc=True),
    )(_sc_body)


def kernel(disease_feats, gene_feats, chemical_feats, species_feats,
           trans_adj_list, P_disease, P_gene, P_chemical, P_species, W_CDD):
    del gene_feats, species_feats, P_gene, P_species
    g0, g1, g2 = _make_tables(disease_feats, chemical_feats,
                              P_disease, P_chemical, W_CDD)
    idx = trans_adj_list.astype(jnp.int32)
    return _sc_gather()(g0, g1, g2, idx[0], idx[1], idx[2])


# speedup vs baseline: 1.8205x; 1.8205x over previous
import functools

import jax
import jax.numpy as jnp
from jax import lax
from jax.experimental import pallas as pl
from jax.experimental.pallas import tpu as pltpu
from jax.experimental.pallas import tpu_sc as plsc

IN_DIM = 128
OUT_DIM = 32
N_NODES = 10000
N_INST = 320000
ALPHA = 0.2

NUM_CORES = 2
NUM_SUBCORES = 16
NW = NUM_CORES * NUM_SUBCORES
B_PER_W = N_INST // NW
CHUNK = 80
NCHUNK = B_PER_W // CHUNK


def _tables_body(dis_ref, chem_ref, pd_ref, pc_ref, w_ref, g0_ref, g1_ref, g2_ref):
    q0 = jnp.dot(pd_ref[...], w_ref[0:32, :], preferred_element_type=jnp.float32)
    q1 = jnp.dot(pd_ref[...], w_ref[32:64, :], preferred_element_type=jnp.float32)
    q2 = jnp.dot(pc_ref[...], w_ref[64:96, :], preferred_element_type=jnp.float32)
    d = dis_ref[...]
    c = chem_ref[...]
    g0_ref[...] = jnp.dot(d, q0, preferred_element_type=jnp.float32)
    g1_ref[...] = jnp.dot(d, q1, preferred_element_type=jnp.float32)
    g2_ref[...] = jnp.dot(c, q2, preferred_element_type=jnp.float32)


def _make_tables(disease_feats, chemical_feats, P_disease, P_chemical, W_CDD):
    blk = 2000
    grid = N_NODES // blk
    return pl.pallas_call(
        _tables_body,
        grid=(grid,),
        in_specs=[
            pl.BlockSpec((blk, IN_DIM), lambda i: (i, 0)),
            pl.BlockSpec((blk, IN_DIM), lambda i: (i, 0)),
            pl.BlockSpec((IN_DIM, OUT_DIM), lambda i: (0, 0)),
            pl.BlockSpec((IN_DIM, OUT_DIM), lambda i: (0, 0)),
            pl.BlockSpec((3 * OUT_DIM, OUT_DIM), lambda i: (0, 0)),
        ],
        out_specs=[
            pl.BlockSpec((blk, OUT_DIM), lambda i: (i, 0)),
            pl.BlockSpec((blk, OUT_DIM), lambda i: (i, 0)),
            pl.BlockSpec((blk, OUT_DIM), lambda i: (i, 0)),
        ],
        out_shape=[jax.ShapeDtypeStruct((N_NODES, OUT_DIM), jnp.float32)] * 3,
    )(disease_feats, chemical_feats, P_disease, P_chemical, W_CDD)


def _sc_body(g0_hbm, g1_hbm, g2_hbm, idx_hbm, out_hbm,
             i_v, r_v, o_v, isem, gsem, osem):
    wid = lax.axis_index("s") * NUM_CORES + lax.axis_index("c")
    base = wid * B_PER_W
    tables = (g0_hbm, g1_hbm, g2_hbm)

    def load_idx(chunk, slot):
        off = base + chunk * CHUNK
        for k in range(3):
            pltpu.async_copy(idx_hbm.at[k, pl.ds(off, CHUNK)], i_v.at[slot, k], isem)

    def wait_idx(slot):
        for k in range(3):
            pltpu.make_async_copy(
                idx_hbm.at[k, pl.ds(base, CHUNK)], i_v.at[slot, k], isem).wait()

    def issue_gathers(slot):
        for k in range(3):
            pltpu.async_copy(tables[k].at[i_v.at[slot, k]], r_v.at[slot, k], gsem)

    def wait_gathers(slot):
        for k in range(3):
            pltpu.make_async_copy(
                tables[k].at[i_v.at[slot, k]], r_v.at[slot, k], gsem).wait()

    lane = lax.iota(jnp.int32, 16)

    def compute(slot):
        @plsc.parallel_loop(0, (CHUNK // 16) * 2, 1)
        def _(b):
            bi = b // 2
            h = lax.rem(b, 2)
            r0 = bi * 16
            s = pl.ds(h * 16, 16)
            v = []
            for i in range(16):
                r = r0 + i
                x = r_v[slot, 0, r, s] + r_v[slot, 1, r, s] + r_v[slot, 2, r, s]
                v.append(jnp.maximum(x, ALPHA * x))
            for st in (1, 2, 4, 8):
                perm = jnp.bitwise_xor(lane, st)
                v = [jnp.where((lane & st) == (i & st), v[i],
                               v[i ^ st].at[perm].get(mode="promise_in_bounds"))
                     for i in range(16)]
            for j in range(16):
                o_v[slot, h * 16 + j, pl.ds(r0, 16)] = v[j]

    def wait_store(slot):
        pltpu.make_async_copy(
            o_v.at[slot],
            out_hbm.at[pl.ds(0, OUT_DIM), pl.ds(base, CHUNK)], osem).wait()

    load_idx(0, 0)
    wait_idx(0)
    issue_gathers(0)
    load_idx(1, 1)

    def step(j, carry):
        slot = lax.rem(j, 2)
        nslot = 1 - slot
        wait_idx(nslot)
        issue_gathers(nslot)
        wait_gathers(slot)

        @pl.when(j < NCHUNK - 2)
        def _():
            load_idx(j + 2, slot)

        compute(slot)

        @pl.when(j > 0)
        def _():
            wait_store(nslot)
        pltpu.async_copy(
            o_v.at[slot],
            out_hbm.at[pl.ds(0, OUT_DIM), pl.ds(base + j * CHUNK, CHUNK)],
            osem)
        return carry

    lax.fori_loop(0, NCHUNK - 1, step, 0)

    last = NCHUNK - 1
    wait_gathers(0)
    compute(0)
    wait_store(1)
    pltpu.async_copy(
        o_v.at[0],
        out_hbm.at[pl.ds(0, OUT_DIM), pl.ds(base + last * CHUNK, CHUNK)],
        osem)
    wait_store(0)


@functools.cache
def _sc_gather():
    return functools.partial(
        pl.kernel,
        out_type=jax.ShapeDtypeStruct((OUT_DIM, N_INST), jnp.float32),
        mesh=plsc.VectorSubcoreMesh(
            core_axis_name="c", subcore_axis_name="s",
            num_cores=NUM_CORES, num_subcores=NUM_SUBCORES),
        scratch_types=[
            pltpu.VMEM((2, 3, CHUNK), jnp.int32),
            pltpu.VMEM((2, 3, CHUNK, OUT_DIM), jnp.float32),
            pltpu.VMEM((2, OUT_DIM, CHUNK), jnp.float32),
            pltpu.SemaphoreType.DMA,
            pltpu.SemaphoreType.DMA,
            pltpu.SemaphoreType.DMA,
        ],
        compiler_params=pltpu.CompilerParams(use_tc_tiling_on_sc=False),
    )(_sc_body)


def kernel(disease_feats, gene_feats, chemical_feats, species_feats,
           trans_adj_list, P_disease, P_gene, P_chemical, P_species, W_CDD):
    del gene_feats, species_feats, P_gene, P_species
    g0, g1, g2 = _make_tables(disease_feats, chemical_feats,
                              P_disease, P_chemical, W_CDD)
    idx = trans_adj_list.astype(jnp.int32)
    out_t = _sc_gather()(g0, g1, g2, idx)
    return jnp.transpose(out_t)
